# baseline (speedup 1.0000x reference)
import functools
import math

import jax
import jax.numpy as jnp
from jax import lax
from jax.experimental import pallas as pl
from jax.experimental.pallas import tpu as pltpu
from jax.experimental.pallas import tpu_sc as plsc

_STRIDES = (8, 16, 32, 64, 128)
_IMG = 512
_NCLS = 21
_NCH = 4 + 1 + _NCLS
_NB = 64
_B = 8
_NC = 2
_NS = 16
_LANES = 16


def _thresholds():
    result = []
    last = _IMG
    for i in range(len(_STRIDES) - 1, -1, -1):
        s = _STRIDES[i]
        px = float(s) / _IMG
        th_max = math.ceil(last / s)
        if th_max % 2:
            th_max += 1
        th_min = th_max // 2
        last = th_min * s
        if i == 0:
            th_min = 1
        result.append((th_min * px, th_max * px))
    return tuple(result[::-1])

_THS = _thresholds()


def _sqrt_nr(a):
    i = lax.bitcast_convert_type(a, jnp.int32)
    i = jnp.int32(0x5F3759DF) - lax.shift_right_logical(i, 1)
    r = lax.bitcast_convert_type(i, jnp.float32)
    for _ in range(3):
        r = r * (1.5 - 0.5 * a * r * r)
    return a * r


def _sc_mapper(boxes_hbm, labels_hbm, o0, o1, o2, o3, o4,
               boxes_v, labels_v, areas_v, stage0, stage1, stage2, stage3,
               stage4, win_v, mn_v):
    wid = lax.axis_index("s") * _NC + lax.axis_index("c")

    pltpu.sync_copy(boxes_hbm, boxes_v)
    pltpu.sync_copy(labels_hbm, labels_v)

    iota = lax.iota(jnp.int32, _LANES)
    iota4 = iota * 4

    def pix_coords(lev, p):
        size = _IMG // _STRIDES[lev]
        log2s = size.bit_length() - 1
        scale = float(_STRIDES[lev]) / _IMG
        px = jnp.bitwise_and(p, size - 1)
        py = lax.shift_right_logical(p, log2s)
        cx = (px.astype(jnp.float32) + 0.5) * scale
        cy = (py.astype(jnp.float32) + 0.5) * scale
        return cx, cy

    def scan_block(img, base, lev, nchunks, woff):
        size = _IMG // _STRIDES[lev]
        th0, th1 = _THS[lev]
        cpr = size // _LANES

        cxs, cys = [], []
        for j in range(nchunks):
            cx, cy = pix_coords(lev, base + j * _LANES + iota)
            cxs.append(cx)
            cys.append(cy)

        bbase = img * (_NB * 4)

        def body(i, carry):
            mnm = list(carry[0])
            war = list(carry[1])
            wix = list(carry[2])
            qv = jnp.full((_LANES,), bbase + i * 4, jnp.int32)
            x1 = plsc.load_gather(boxes_v, [qv])
            y1 = plsc.load_gather(boxes_v, [qv + 1])
            x2 = plsc.load_gather(boxes_v, [qv + 2])
            y2 = plsc.load_gather(boxes_v, [qv + 3])
            ar = plsc.load_gather(areas_v, [jnp.full((_LANES,), i, jnp.int32)])
            fi = i.astype(jnp.float32)
            tbs = []
            if cpr >= 1:
                for r in range(nchunks // cpr):
                    cy = cys[r * cpr]
                    t = cy - y1
                    b = y2 - cy
                    tbs.append((jnp.minimum(t, b), jnp.maximum(t, b)))
            for j in range(nchunks):
                if cpr >= 1:
                    mint, maxt = tbs[j // cpr]
                else:
                    t = cys[j] - y1
                    b = y2 - cys[j]
                    mint = jnp.minimum(t, b)
                    maxt = jnp.maximum(t, b)
                l = cxs[j] - x1
                rr = x2 - cxs[j]
                mn = jnp.minimum(jnp.minimum(l, rr), mint)
                mx = jnp.maximum(jnp.maximum(l, rr), maxt)
                mnm[j] = jnp.maximum(mnm[j], mn)
                pred = (mn >= 0.0) & (mx > th0) & (mx <= th1)
                better = pred & (ar <= war[j])
                war[j] = jnp.where(better, ar, war[j])
                wix[j] = jnp.where(better, fi, wix[j])
            return (tuple(mnm), tuple(war), tuple(wix))

        neg1 = jnp.full((_LANES,), -1.0, jnp.float32)
        big = jnp.full((_LANES,), 3.0e38, jnp.float32)
        init = (tuple(neg1 for _ in range(nchunks)),
                tuple(big for _ in range(nchunks)),
                tuple(neg1 for _ in range(nchunks)))
        mnm, _, wix = lax.fori_loop(0, _NB, body, init)
        for j in range(nchunks):
            d = pl.ds((woff + j) * _LANES, _LANES)
            win_v[d] = wix[j]
            mn_v[d] = mnm[j]

    def emit_chunks(img, band, lev, nchunks, store):
        bbase = img * (_NB * 4)

        def ebody(c, carry):
            d = pl.ds(c * _LANES, _LANES)
            widxf = win_v[d]
            mnmax = mn_v[d]
            cx, cy = pix_coords(lev, band + c * _LANES + iota)
            anyfg = mnmax >= 0.0
            haswin = widxf >= 0.0
            wi = jnp.where(haswin, widxf, 0.0).astype(jnp.int32)
            wq = wi * 4 + jnp.full((_LANES,), bbase, jnp.int32)
            x1w = plsc.load_gather(boxes_v, [wq])
            y1w = plsc.load_gather(boxes_v, [wq + 1])
            x2w = plsc.load_gather(boxes_v, [wq + 2])
            y2w = plsc.load_gather(boxes_v, [wq + 3])
            labw = plsc.load_gather(
                labels_v, [wi + jnp.full((_LANES,), img * _NB, jnp.int32)])
            lr = cx - x1w
            tr = cy - y1w
            rr = x2w - cx
            br = y2w - cy
            arg = ((jnp.minimum(lr, rr) / jnp.maximum(lr, rr)) *
                   (jnp.minimum(tr, br) / jnp.maximum(tr, br)))
            pos = haswin & (arg > 0.0)
            safe = jnp.where(pos, arg, 1.0)
            cen = jnp.where(pos, _sqrt_nr(safe), 0.0)
            labm = jnp.where(haswin, labw, 0)

            store(0, c, jnp.where(haswin, lr, 0.0))
            store(1, c, jnp.where(haswin, tr, 0.0))
            store(2, c, jnp.where(haswin, rr, 0.0))
            store(3, c, jnp.where(haswin, br, 0.0))
            store(4, c, cen)
            store(5, c, jnp.where(anyfg, 0.0, 1.0))
            for ch in range(1, _NCLS):
                store(5 + ch, c, jnp.where(labm == ch, 1.0, 0.0))
            return carry

        lax.fori_loop(0, nchunks, ebody, 0)

    def img_body(img, _):
        for k in range(_NB // _LANES):
            qb = img * (_NB * 4) + k * _LANES * 4
            qv = iota4 + jnp.full((_LANES,), qb, jnp.int32)
            x1v = plsc.load_gather(boxes_v, [qv])
            y1v = plsc.load_gather(boxes_v, [qv + 1])
            x2v = plsc.load_gather(boxes_v, [qv + 2])
            y2v = plsc.load_gather(boxes_v, [qv + 3])
            areas_v[pl.ds(k * _LANES, _LANES)] = (x2v - x1v) * (y2v - y1v)

        b0 = jnp.remainder(wid + 24 * img, 32)

        @pl.when(b0 < 8)
        def _():
            band = b0 * 512

            def blk(kb, carry):
                scan_block(img, band + kb * 256, 0, 16, kb * 16)
                return carry
            lax.fori_loop(0, 2, blk, 0)

            def st0(ch, c, v):
                stage0[ch, lax.shift_right_logical(c, 2),
                       pl.ds(jnp.bitwise_and(c, 3) * _LANES, _LANES)] = v
            emit_chunks(img, band, 0, 32, st0)
            pltpu.sync_copy(stage0, o0.at[img, :, pl.ds(b0 * 8, 8), :])

        b1 = jnp.remainder(wid + 28 * img, 32)

        @pl.when(b1 < 4)
        def _():
            band = b1 * 256

            def blk(kb, carry):
                scan_block(img, band + kb * 128, 1, 8, kb * 8)
                return carry
            lax.fori_loop(0, 2, blk, 0)

            def st1(ch, c, v):
                stage1[ch, lax.shift_right_logical(c, 1),
                       pl.ds(jnp.bitwise_and(c, 1) * _LANES, _LANES)] = v
            emit_chunks(img, band, 1, 16, st1)
            pltpu.sync_copy(stage1, o1.at[img, :, pl.ds(b1 * 8, 8), :])

        b2 = jnp.remainder(wid + 30 * img, 32)

        @pl.when(b2 < 2)
        def _():
            band = b2 * 128
            scan_block(img, band, 2, 8, 0)

            def st2(ch, c, v):
                stage2[ch, c, pl.ds(0, _LANES)] = v
            emit_chunks(img, band, 2, 8, st2)
            pltpu.sync_copy(stage2, o2.at[img, :, pl.ds(b2 * 8, 8), :])

        @pl.when(wid == 16 + img)
        def _():
            scan_block(img, 0, 3, 4, 0)

            def st3(ch, c, v):
                stage3[ch, pl.ds(c * _LANES, _LANES)] = v
            emit_chunks(img, 0, 3, 4, st3)
            pltpu.sync_copy(stage3, o3.at[img])

        @pl.when(wid == 24 + img)
        def _():
            scan_block(img, 0, 4, 1, 0)

            def st4(ch, c, v):
                stage4[ch, pl.ds(0, _LANES)] = v
            emit_chunks(img, 0, 4, 1, st4)
            pltpu.sync_copy(stage4, o4.at[img])

        return 0

    lax.fori_loop(0, _B, img_body, 0)


def kernel(boxes, labels):
    bflat = boxes.reshape(-1)
    lflat = labels.reshape(-1)

    out_type = (
        jax.ShapeDtypeStruct((_B, _NCH, 64, 64), jnp.float32),
        jax.ShapeDtypeStruct((_B, _NCH, 32, 32), jnp.float32),
        jax.ShapeDtypeStruct((_B, _NCH, 16, 16), jnp.float32),
        jax.ShapeDtypeStruct((_B, _NCH, 64), jnp.float32),
        jax.ShapeDtypeStruct((_B, _NCH, 16), jnp.float32),
    )

    mesh = plsc.VectorSubcoreMesh(core_axis_name="c", subcore_axis_name="s",
                                  num_cores=_NC, num_subcores=_NS)
    run = pl.kernel(
        _sc_mapper,
        out_type=out_type,
        mesh=mesh,
        compiler_params=pltpu.CompilerParams(needs_layout_passes=False),
        scratch_types=[
            pltpu.VMEM((_B * _NB * 4,), jnp.float32),
            pltpu.VMEM((_B * _NB,), jnp.int32),
            pltpu.VMEM((_NB,), jnp.float32),
            pltpu.VMEM((_NCH, 8, 64), jnp.float32),
            pltpu.VMEM((_NCH, 8, 32), jnp.float32),
            pltpu.VMEM((_NCH, 8, 16), jnp.float32),
            pltpu.VMEM((_NCH, 64), jnp.float32),
            pltpu.VMEM((_NCH, 16), jnp.float32),
            pltpu.VMEM((512,), jnp.float32),
            pltpu.VMEM((512,), jnp.float32),
        ],
    )
    o0, o1, o2, o3, o4 = run(bflat, lflat)
    return (o0, o1, o2,
            o3.reshape(_B, _NCH, 8, 8),
            o4.reshape(_B, _NCH, 4, 4))


# speedup vs baseline: 1.8697x; 1.8697x over previous
import functools
import math

import jax
import jax.numpy as jnp
from jax import lax
from jax.experimental import pallas as pl
from jax.experimental.pallas import tpu as pltpu
from jax.experimental.pallas import tpu_sc as plsc

_STRIDES = (8, 16, 32, 64, 128)
_IMG = 512
_NCLS = 21
_NCH = 4 + 1 + _NCLS
_NB = 64
_B = 8
_NC = 2
_NS = 16
_LANES = 16


def _thresholds():
    result = []
    last = _IMG
    for i in range(len(_STRIDES) - 1, -1, -1):
        s = _STRIDES[i]
        px = float(s) / _IMG
        th_max = math.ceil(last / s)
        if th_max % 2:
            th_max += 1
        th_min = th_max // 2
        last = th_min * s
        if i == 0:
            th_min = 1
        result.append((th_min * px, th_max * px))
    return tuple(result[::-1])

_THS = _thresholds()


def _sqrt_nr(a):
    i = lax.bitcast_convert_type(a, jnp.int32)
    i = jnp.int32(0x5F3759DF) - lax.shift_right_logical(i, 1)
    r = lax.bitcast_convert_type(i, jnp.float32)
    for _ in range(3):
        r = r * (1.5 - 0.5 * a * r * r)
    return a * r


def _sc_mapper(boxes_hbm, labels_hbm, o0, o1, o2, o3, o4,
               boxes_v, labels_v, areas_v, stage0, stage1, stage2, stage3,
               stage4, win_v, mn_v):
    wid = lax.axis_index("s") * _NC + lax.axis_index("c")

    pltpu.sync_copy(boxes_hbm, boxes_v)
    pltpu.sync_copy(labels_hbm, labels_v)

    iota = lax.iota(jnp.int32, _LANES)
    iota4 = iota * 4

    def pix_coords(lev, p):
        size = _IMG // _STRIDES[lev]
        log2s = size.bit_length() - 1
        scale = float(_STRIDES[lev]) / _IMG
        px = jnp.bitwise_and(p, size - 1)
        py = lax.shift_right_logical(p, log2s)
        cx = (px.astype(jnp.float32) + 0.5) * scale
        cy = (py.astype(jnp.float32) + 0.5) * scale
        return cx, cy

    def scan_block(img, base, lev, nchunks, woff):
        size = _IMG // _STRIDES[lev]
        th0, th1 = _THS[lev]
        cpr = size // _LANES

        cxs, cys = [], []
        for j in range(nchunks):
            cx, cy = pix_coords(lev, base + j * _LANES + iota)
            cxs.append(cx)
            cys.append(cy)

        bbase = img * (_NB * 4)

        def body(i, carry):
            mnm = list(carry[0])
            war = list(carry[1])
            wix = list(carry[2])
            qv = jnp.full((_LANES,), bbase + i * 4, jnp.int32)
            x1 = plsc.load_gather(boxes_v, [qv])
            y1 = plsc.load_gather(boxes_v, [qv + 1])
            x2 = plsc.load_gather(boxes_v, [qv + 2])
            y2 = plsc.load_gather(boxes_v, [qv + 3])
            ar = plsc.load_gather(areas_v, [jnp.full((_LANES,), i, jnp.int32)])
            fi = i.astype(jnp.float32)
            tbs = []
            if cpr >= 1:
                for r in range(nchunks // cpr):
                    cy = cys[r * cpr]
                    t = cy - y1
                    b = y2 - cy
                    tbs.append((jnp.minimum(t, b), jnp.maximum(t, b)))
            for j in range(nchunks):
                if cpr >= 1:
                    mint, maxt = tbs[j // cpr]
                else:
                    t = cys[j] - y1
                    b = y2 - cys[j]
                    mint = jnp.minimum(t, b)
                    maxt = jnp.maximum(t, b)
                l = cxs[j] - x1
                rr = x2 - cxs[j]
                mn = jnp.minimum(jnp.minimum(l, rr), mint)
                mx = jnp.maximum(jnp.maximum(l, rr), maxt)
                mnm[j] = jnp.maximum(mnm[j], mn)
                pred = (mn >= 0.0) & (mx > th0) & (mx <= th1)
                better = pred & (ar <= war[j])
                war[j] = jnp.where(better, ar, war[j])
                wix[j] = jnp.where(better, fi, wix[j])
            return (tuple(mnm), tuple(war), tuple(wix))

        neg1 = jnp.full((_LANES,), -1.0, jnp.float32)
        big = jnp.full((_LANES,), 3.0e38, jnp.float32)
        init = (tuple(neg1 for _ in range(nchunks)),
                tuple(big for _ in range(nchunks)),
                tuple(neg1 for _ in range(nchunks)))
        mnm, _, wix = lax.fori_loop(0, _NB, body, init)
        for j in range(nchunks):
            d = pl.ds((woff + j) * _LANES, _LANES)
            win_v[d] = wix[j]
            mn_v[d] = mnm[j]

    def emit_chunks(img, band, lev, nchunks, store):
        bbase = img * (_NB * 4)

        def ebody(c, carry):
            d = pl.ds(c * _LANES, _LANES)
            widxf = win_v[d]
            mnmax = mn_v[d]
            cx, cy = pix_coords(lev, band + c * _LANES + iota)
            anyfg = mnmax >= 0.0
            haswin = widxf >= 0.0
            wi = jnp.where(haswin, widxf, 0.0).astype(jnp.int32)
            wq = wi * 4 + jnp.full((_LANES,), bbase, jnp.int32)
            x1w = plsc.load_gather(boxes_v, [wq])
            y1w = plsc.load_gather(boxes_v, [wq + 1])
            x2w = plsc.load_gather(boxes_v, [wq + 2])
            y2w = plsc.load_gather(boxes_v, [wq + 3])
            labw = plsc.load_gather(
                labels_v, [wi + jnp.full((_LANES,), img * _NB, jnp.int32)])
            lr = cx - x1w
            tr = cy - y1w
            rr = x2w - cx
            br = y2w - cy
            arg = ((jnp.minimum(lr, rr) / jnp.maximum(lr, rr)) *
                   (jnp.minimum(tr, br) / jnp.maximum(tr, br)))
            pos = haswin & (arg > 0.0)
            safe = jnp.where(pos, arg, 1.0)
            cen = jnp.where(pos, _sqrt_nr(safe), 0.0)
            labm = jnp.where(haswin, labw, 0)

            store(0, c, jnp.where(haswin, lr, 0.0))
            store(1, c, jnp.where(haswin, tr, 0.0))
            store(2, c, jnp.where(haswin, rr, 0.0))
            store(3, c, jnp.where(haswin, br, 0.0))
            store(4, c, cen)
            store(5, c, jnp.where(anyfg, 0.0, 1.0))
            for ch in range(1, _NCLS):
                store(5 + ch, c, jnp.where(labm == ch, 1.0, 0.0))
            return carry

        lax.fori_loop(0, nchunks, ebody, 0)

    def img_body(img, _):
        for k in range(_NB // _LANES):
            qb = img * (_NB * 4) + k * _LANES * 4
            qv = iota4 + jnp.full((_LANES,), qb, jnp.int32)
            x1v = plsc.load_gather(boxes_v, [qv])
            y1v = plsc.load_gather(boxes_v, [qv + 1])
            x2v = plsc.load_gather(boxes_v, [qv + 2])
            y2v = plsc.load_gather(boxes_v, [qv + 3])
            areas_v[pl.ds(k * _LANES, _LANES)] = (x2v - x1v) * (y2v - y1v)

        b0 = jnp.remainder(wid + 24 * img, 32)

        @pl.when(b0 < 8)
        def _():
            band = b0 * 512

            def blk(kb, carry):
                scan_block(img, band + kb * 128, 0, 8, kb * 8)
                return carry
            lax.fori_loop(0, 4, blk, 0)

            def st0(ch, c, v):
                stage0[ch, lax.shift_right_logical(c, 2),
                       pl.ds(jnp.bitwise_and(c, 3) * _LANES, _LANES)] = v
            emit_chunks(img, band, 0, 32, st0)
            pltpu.sync_copy(stage0, o0.at[img, :, pl.ds(b0 * 8, 8), :])

        b1 = jnp.remainder(wid + 28 * img, 32)

        @pl.when(b1 < 4)
        def _():
            band = b1 * 256

            def blk(kb, carry):
                scan_block(img, band + kb * 128, 1, 8, kb * 8)
                return carry
            lax.fori_loop(0, 2, blk, 0)

            def st1(ch, c, v):
                stage1[ch, lax.shift_right_logical(c, 1),
                       pl.ds(jnp.bitwise_and(c, 1) * _LANES, _LANES)] = v
            emit_chunks(img, band, 1, 16, st1)
            pltpu.sync_copy(stage1, o1.at[img, :, pl.ds(b1 * 8, 8), :])

        b2 = jnp.remainder(wid + 30 * img, 32)

        @pl.when(b2 < 2)
        def _():
            band = b2 * 128
            scan_block(img, band, 2, 8, 0)

            def st2(ch, c, v):
                stage2[ch, c, pl.ds(0, _LANES)] = v
            emit_chunks(img, band, 2, 8, st2)
            pltpu.sync_copy(stage2, o2.at[img, :, pl.ds(b2 * 8, 8), :])

        @pl.when(wid == 16 + img)
        def _():
            scan_block(img, 0, 3, 4, 0)

            def st3(ch, c, v):
                stage3[ch, pl.ds(c * _LANES, _LANES)] = v
            emit_chunks(img, 0, 3, 4, st3)
            pltpu.sync_copy(stage3, o3.at[img])

        @pl.when(wid == 24 + img)
        def _():
            scan_block(img, 0, 4, 1, 0)

            def st4(ch, c, v):
                stage4[ch, pl.ds(0, _LANES)] = v
            emit_chunks(img, 0, 4, 1, st4)
            pltpu.sync_copy(stage4, o4.at[img])

        return 0

    lax.fori_loop(0, _B, img_body, 0)


def kernel(boxes, labels):
    bflat = boxes.reshape(-1)
    lflat = labels.reshape(-1)

    out_type = (
        jax.ShapeDtypeStruct((_B, _NCH, 64, 64), jnp.float32),
        jax.ShapeDtypeStruct((_B, _NCH, 32, 32), jnp.float32),
        jax.ShapeDtypeStruct((_B, _NCH, 16, 16), jnp.float32),
        jax.ShapeDtypeStruct((_B, _NCH, 64), jnp.float32),
        jax.ShapeDtypeStruct((_B, _NCH, 16), jnp.float32),
    )

    mesh = plsc.VectorSubcoreMesh(core_axis_name="c", subcore_axis_name="s",
                                  num_cores=_NC, num_subcores=_NS)
    run = pl.kernel(
        _sc_mapper,
        out_type=out_type,
        mesh=mesh,
        compiler_params=pltpu.CompilerParams(needs_layout_passes=False),
        scratch_types=[
            pltpu.VMEM((_B * _NB * 4,), jnp.float32),
            pltpu.VMEM((_B * _NB,), jnp.int32),
            pltpu.VMEM((_NB,), jnp.float32),
            pltpu.VMEM((_NCH, 8, 64), jnp.float32),
            pltpu.VMEM((_NCH, 8, 32), jnp.float32),
            pltpu.VMEM((_NCH, 8, 16), jnp.float32),
            pltpu.VMEM((_NCH, 64), jnp.float32),
            pltpu.VMEM((_NCH, 16), jnp.float32),
            pltpu.VMEM((512,), jnp.float32),
            pltpu.VMEM((512,), jnp.float32),
        ],
    )
    o0, o1, o2, o3, o4 = run(bflat, lflat)
    return (o0, o1, o2,
            o3.reshape(_B, _NCH, 8, 8),
            o4.reshape(_B, _NCH, 4, 4))


# speedup vs baseline: 2.1330x; 1.1408x over previous
import functools
import math

import jax
import jax.numpy as jnp
from jax import lax
from jax.experimental import pallas as pl
from jax.experimental.pallas import tpu as pltpu
from jax.experimental.pallas import tpu_sc as plsc

_STRIDES = (8, 16, 32, 64, 128)
_IMG = 512
_NCLS = 21
_NCH = 4 + 1 + _NCLS
_NB = 64
_B = 8
_NC = 2
_NS = 16
_LANES = 16


def _thresholds():
    result = []
    last = _IMG
    for i in range(len(_STRIDES) - 1, -1, -1):
        s = _STRIDES[i]
        px = float(s) / _IMG
        th_max = math.ceil(last / s)
        if th_max % 2:
            th_max += 1
        th_min = th_max // 2
        last = th_min * s
        if i == 0:
            th_min = 1
        result.append((th_min * px, th_max * px))
    return tuple(result[::-1])

_THS = _thresholds()


def _sqrt_nr(a):
    i = lax.bitcast_convert_type(a, jnp.int32)
    i = jnp.int32(0x5F3759DF) - lax.shift_right_logical(i, 1)
    r = lax.bitcast_convert_type(i, jnp.float32)
    for _ in range(3):
        r = r * (1.5 - 0.5 * a * r * r)
    return a * r


def _sc_mapper(boxes_hbm, labels_hbm, o0, o1, o2, o3, o4,
               boxes_v, labels_v, areas_v, stage0, stage1, stage2, stage3,
               stage4, win_v, mn_v, glist_v):
    wid = lax.axis_index("s") * _NC + lax.axis_index("c")

    pltpu.sync_copy(boxes_hbm, boxes_v)
    pltpu.sync_copy(labels_hbm, labels_v)

    iota = lax.iota(jnp.int32, _LANES)
    iota4 = iota * 4

    def pix_coords(lev, p):
        size = _IMG // _STRIDES[lev]
        log2s = size.bit_length() - 1
        scale = float(_STRIDES[lev]) / _IMG
        px = jnp.bitwise_and(p, size - 1)
        py = lax.shift_right_logical(p, log2s)
        cx = (px.astype(jnp.float32) + 0.5) * scale
        cy = (py.astype(jnp.float32) + 0.5) * scale
        return cx, cy

    def scan_block(img, base, lev, nchunks, woff, gcnt):
        size = _IMG // _STRIDES[lev]
        th0, th1 = _THS[lev]
        cpr = size // _LANES

        cxs, cys = [], []
        for j in range(nchunks):
            cx, cy = pix_coords(lev, base + j * _LANES + iota)
            cxs.append(cx)
            cys.append(cy)

        bbase = img * (_NB * 4)

        def mints_of(y1, y2):
            tbs = []
            if cpr >= 1:
                for r in range(nchunks // cpr):
                    cy = cys[r * cpr]
                    t = cy - y1
                    b = y2 - cy
                    tbs.append((jnp.minimum(t, b), jnp.maximum(t, b)))

            def tb_of(j):
                if cpr >= 1:
                    return tbs[j // cpr]
                t = cys[j] - y1
                b = y2 - cys[j]
                return jnp.minimum(t, b), jnp.maximum(t, b)
            return tb_of

        def fg_body(i, carry):
            mnm = list(carry)
            qv = jnp.full((_LANES,), bbase + i * 4, jnp.int32)
            x1 = plsc.load_gather(boxes_v, [qv])
            y1 = plsc.load_gather(boxes_v, [qv + 1])
            x2 = plsc.load_gather(boxes_v, [qv + 2])
            y2 = plsc.load_gather(boxes_v, [qv + 3])
            tb_of = mints_of(y1, y2)
            for j in range(nchunks):
                mint, _ = tb_of(j)
                l = cxs[j] - x1
                rr = x2 - cxs[j]
                mn = jnp.minimum(jnp.minimum(l, rr), mint)
                mnm[j] = jnp.maximum(mnm[j], mn)
            return tuple(mnm)

        def win_body(i, carry):
            war = list(carry[0])
            wix = list(carry[1])
            bi = plsc.load_gather(
                glist_v, [jnp.full((_LANES,), lev * _NB, jnp.int32) + i])
            qv = bi * 4 + jnp.full((_LANES,), bbase, jnp.int32)
            x1 = plsc.load_gather(boxes_v, [qv])
            y1 = plsc.load_gather(boxes_v, [qv + 1])
            x2 = plsc.load_gather(boxes_v, [qv + 2])
            y2 = plsc.load_gather(boxes_v, [qv + 3])
            ar = plsc.load_gather(areas_v, [bi])
            fi = bi.astype(jnp.float32)
            tb_of = mints_of(y1, y2)
            for j in range(nchunks):
                mint, maxt = tb_of(j)
                l = cxs[j] - x1
                rr = x2 - cxs[j]
                mn = jnp.minimum(jnp.minimum(l, rr), mint)
                mx = jnp.maximum(jnp.maximum(l, rr), maxt)
                pred = (mn >= 0.0) & (mx > th0) & (mx <= th1)
                better = pred & (ar <= war[j])
                war[j] = jnp.where(better, ar, war[j])
                wix[j] = jnp.where(better, fi, wix[j])
            return (tuple(war), tuple(wix))

        neg1 = jnp.full((_LANES,), -1.0, jnp.float32)
        big = jnp.full((_LANES,), 3.0e38, jnp.float32)
        mnm = lax.fori_loop(0, _NB, fg_body,
                            tuple(neg1 for _ in range(nchunks)))
        _, wix = lax.fori_loop(
            0, gcnt, win_body,
            (tuple(big for _ in range(nchunks)),
             tuple(neg1 for _ in range(nchunks))))
        for j in range(nchunks):
            d = pl.ds((woff + j) * _LANES, _LANES)
            win_v[d] = wix[j]
            mn_v[d] = mnm[j]

    def emit_chunks(img, band, lev, nchunks, store):
        bbase = img * (_NB * 4)

        def ebody(c, carry):
            d = pl.ds(c * _LANES, _LANES)
            widxf = win_v[d]
            mnmax = mn_v[d]
            cx, cy = pix_coords(lev, band + c * _LANES + iota)
            anyfg = mnmax >= 0.0
            haswin = widxf >= 0.0
            wi = jnp.where(haswin, widxf, 0.0).astype(jnp.int32)
            wq = wi * 4 + jnp.full((_LANES,), bbase, jnp.int32)
            x1w = plsc.load_gather(boxes_v, [wq])
            y1w = plsc.load_gather(boxes_v, [wq + 1])
            x2w = plsc.load_gather(boxes_v, [wq + 2])
            y2w = plsc.load_gather(boxes_v, [wq + 3])
            labw = plsc.load_gather(
                labels_v, [wi + jnp.full((_LANES,), img * _NB, jnp.int32)])
            lr = cx - x1w
            tr = cy - y1w
            rr = x2w - cx
            br = y2w - cy
            arg = ((jnp.minimum(lr, rr) / jnp.maximum(lr, rr)) *
                   (jnp.minimum(tr, br) / jnp.maximum(tr, br)))
            pos = haswin & (arg > 0.0)
            safe = jnp.where(pos, arg, 1.0)
            cen = jnp.where(pos, _sqrt_nr(safe), 0.0)
            labm = jnp.where(haswin, labw, 0)

            store(0, c, jnp.where(haswin, lr, 0.0))
            store(1, c, jnp.where(haswin, tr, 0.0))
            store(2, c, jnp.where(haswin, rr, 0.0))
            store(3, c, jnp.where(haswin, br, 0.0))
            store(4, c, cen)
            store(5, c, jnp.where(anyfg, 0.0, 1.0))
            for ch in range(1, _NCLS):
                store(5 + ch, c, jnp.where(labm == ch, 1.0, 0.0))
            return carry

        lax.fori_loop(0, nchunks, ebody, 0)

    def img_body(img, _):
        mwhs = []
        for k in range(_NB // _LANES):
            qb = img * (_NB * 4) + k * _LANES * 4
            qv = iota4 + jnp.full((_LANES,), qb, jnp.int32)
            x1v = plsc.load_gather(boxes_v, [qv])
            y1v = plsc.load_gather(boxes_v, [qv + 1])
            x2v = plsc.load_gather(boxes_v, [qv + 2])
            y2v = plsc.load_gather(boxes_v, [qv + 3])
            areas_v[pl.ds(k * _LANES, _LANES)] = (x2v - x1v) * (y2v - y1v)
            mwhs.append(jnp.maximum(x2v - x1v, y2v - y1v))

        geps = 1.0e-5
        gcnts = []
        for lev in range(5):
            th0, th1 = _THS[lev]
            off = jnp.int32(0)
            for k in range(_NB // _LANES):
                m = (mwhs[k] > th0 - geps) & (mwhs[k] <= 2.0 * th1 + geps)
                plsc.store_compressed(
                    glist_v.at[pl.ds(lev * _NB + off, _LANES)],
                    iota + k * _LANES, mask=m)
                off = off + plsc.all_reduce_population_count(m)[0]
            gcnts.append(off)

        b0 = jnp.remainder(wid + 24 * img, 32)

        @pl.when(b0 < 8)
        def _():
            band = b0 * 512

            def blk(kb, carry):
                scan_block(img, band + kb * 128, 0, 8, kb * 8, gcnts[0])
                return carry
            lax.fori_loop(0, 4, blk, 0)

            def st0(ch, c, v):
                stage0[ch, lax.shift_right_logical(c, 2),
                       pl.ds(jnp.bitwise_and(c, 3) * _LANES, _LANES)] = v
            emit_chunks(img, band, 0, 32, st0)
            pltpu.sync_copy(stage0, o0.at[img, :, pl.ds(b0 * 8, 8), :])

        b1 = jnp.remainder(wid + 28 * img, 32)

        @pl.when(b1 < 4)
        def _():
            band = b1 * 256

            def blk(kb, carry):
                scan_block(img, band + kb * 128, 1, 8, kb * 8, gcnts[1])
                return carry
            lax.fori_loop(0, 2, blk, 0)

            def st1(ch, c, v):
                stage1[ch, lax.shift_right_logical(c, 1),
                       pl.ds(jnp.bitwise_and(c, 1) * _LANES, _LANES)] = v
            emit_chunks(img, band, 1, 16, st1)
            pltpu.sync_copy(stage1, o1.at[img, :, pl.ds(b1 * 8, 8), :])

        b2 = jnp.remainder(wid + 30 * img, 32)

        @pl.when(b2 < 2)
        def _():
            band = b2 * 128
            scan_block(img, band, 2, 8, 0, gcnts[2])

            def st2(ch, c, v):
                stage2[ch, c, pl.ds(0, _LANES)] = v
            emit_chunks(img, band, 2, 8, st2)
            pltpu.sync_copy(stage2, o2.at[img, :, pl.ds(b2 * 8, 8), :])

        @pl.when(wid == 16 + img)
        def _():
            scan_block(img, 0, 3, 4, 0, gcnts[3])

            def st3(ch, c, v):
                stage3[ch, pl.ds(c * _LANES, _LANES)] = v
            emit_chunks(img, 0, 3, 4, st3)
            pltpu.sync_copy(stage3, o3.at[img])

        @pl.when(wid == 24 + img)
        def _():
            scan_block(img, 0, 4, 1, 0, gcnts[4])

            def st4(ch, c, v):
                stage4[ch, pl.ds(0, _LANES)] = v
            emit_chunks(img, 0, 4, 1, st4)
            pltpu.sync_copy(stage4, o4.at[img])

        return 0

    lax.fori_loop(0, _B, img_body, 0)


def kernel(boxes, labels):
    bflat = boxes.reshape(-1)
    lflat = labels.reshape(-1)

    out_type = (
        jax.ShapeDtypeStruct((_B, _NCH, 64, 64), jnp.float32),
        jax.ShapeDtypeStruct((_B, _NCH, 32, 32), jnp.float32),
        jax.ShapeDtypeStruct((_B, _NCH, 16, 16), jnp.float32),
        jax.ShapeDtypeStruct((_B, _NCH, 64), jnp.float32),
        jax.ShapeDtypeStruct((_B, _NCH, 16), jnp.float32),
    )

    mesh = plsc.VectorSubcoreMesh(core_axis_name="c", subcore_axis_name="s",
                                  num_cores=_NC, num_subcores=_NS)
    run = pl.kernel(
        _sc_mapper,
        out_type=out_type,
        mesh=mesh,
        compiler_params=pltpu.CompilerParams(needs_layout_passes=False),
        scratch_types=[
            pltpu.VMEM((_B * _NB * 4,), jnp.float32),
            pltpu.VMEM((_B * _NB,), jnp.int32),
            pltpu.VMEM((_NB,), jnp.float32),
            pltpu.VMEM((_NCH, 8, 64), jnp.float32),
            pltpu.VMEM((_NCH, 8, 32), jnp.float32),
            pltpu.VMEM((_NCH, 8, 16), jnp.float32),
            pltpu.VMEM((_NCH, 64), jnp.float32),
            pltpu.VMEM((_NCH, 16), jnp.float32),
            pltpu.VMEM((512,), jnp.float32),
            pltpu.VMEM((512,), jnp.float32),
            pltpu.VMEM((5 * _NB + _LANES,), jnp.int32),
        ],
    )
    o0, o1, o2, o3, o4 = run(bflat, lflat)
    return (o0, o1, o2,
            o3.reshape(_B, _NCH, 8, 8),
            o4.reshape(_B, _NCH, 4, 4))


# speedup vs baseline: 2.2379x; 1.0492x over previous
import functools
import math

import jax
import jax.numpy as jnp
from jax import lax
from jax.experimental import pallas as pl
from jax.experimental.pallas import tpu as pltpu
from jax.experimental.pallas import tpu_sc as plsc

_STRIDES = (8, 16, 32, 64, 128)
_IMG = 512
_NCLS = 21
_NCH = 4 + 1 + _NCLS
_NB = 64
_B = 8
_NC = 2
_NS = 16
_LANES = 16


def _thresholds():
    result = []
    last = _IMG
    for i in range(len(_STRIDES) - 1, -1, -1):
        s = _STRIDES[i]
        px = float(s) / _IMG
        th_max = math.ceil(last / s)
        if th_max % 2:
            th_max += 1
        th_min = th_max // 2
        last = th_min * s
        if i == 0:
            th_min = 1
        result.append((th_min * px, th_max * px))
    return tuple(result[::-1])

_THS = _thresholds()


def _sqrt_nr(a):
    i = lax.bitcast_convert_type(a, jnp.int32)
    i = jnp.int32(0x5F3759DF) - lax.shift_right_logical(i, 1)
    r = lax.bitcast_convert_type(i, jnp.float32)
    for _ in range(3):
        r = r * (1.5 - 0.5 * a * r * r)
    return a * r


def _sc_mapper(boxes_hbm, labels_hbm, o0, o1, o2, o3, o4,
               boxes_v, labels_v, areas_v, stage0, stage1, stage2, stage3,
               stage4, win_v, mn_v, glist_v, blist_v):
    wid = lax.axis_index("s") * _NC + lax.axis_index("c")

    pltpu.sync_copy(boxes_hbm, boxes_v)
    pltpu.sync_copy(labels_hbm, labels_v)

    iota = lax.iota(jnp.int32, _LANES)
    iota4 = iota * 4

    def pix_coords(lev, p):
        size = _IMG // _STRIDES[lev]
        log2s = size.bit_length() - 1
        scale = float(_STRIDES[lev]) / _IMG
        px = jnp.bitwise_and(p, size - 1)
        py = lax.shift_right_logical(p, log2s)
        cx = (px.astype(jnp.float32) + 0.5) * scale
        cy = (py.astype(jnp.float32) + 0.5) * scale
        return cx, cy

    def scan_block(img, base, lev, nchunks, woff, gcnt, fg_src=None,
                   fg_cnt=None, win_off=None):
        size = _IMG // _STRIDES[lev]
        th0, th1 = _THS[lev]
        cpr = size // _LANES

        cxs, cys = [], []
        for j in range(nchunks):
            cx, cy = pix_coords(lev, base + j * _LANES + iota)
            cxs.append(cx)
            cys.append(cy)

        bbase = img * (_NB * 4)

        def mints_of(y1, y2):
            tbs = []
            if cpr >= 1:
                for r in range(nchunks // cpr):
                    cy = cys[r * cpr]
                    t = cy - y1
                    b = y2 - cy
                    tbs.append((jnp.minimum(t, b), jnp.maximum(t, b)))

            def tb_of(j):
                if cpr >= 1:
                    return tbs[j // cpr]
                t = cys[j] - y1
                b = y2 - cys[j]
                return jnp.minimum(t, b), jnp.maximum(t, b)
            return tb_of

        def fg_body(i, carry):
            mnm = list(carry)
            if fg_src is None:
                qv = jnp.full((_LANES,), bbase + i * 4, jnp.int32)
            else:
                bi = plsc.load_gather(
                    blist_v, [jnp.full((_LANES,), fg_src, jnp.int32) + i])
                qv = bi * 4 + jnp.full((_LANES,), bbase, jnp.int32)
            x1 = plsc.load_gather(boxes_v, [qv])
            y1 = plsc.load_gather(boxes_v, [qv + 1])
            x2 = plsc.load_gather(boxes_v, [qv + 2])
            y2 = plsc.load_gather(boxes_v, [qv + 3])
            tb_of = mints_of(y1, y2)
            for j in range(nchunks):
                mint, _ = tb_of(j)
                l = cxs[j] - x1
                rr = x2 - cxs[j]
                mn = jnp.minimum(jnp.minimum(l, rr), mint)
                mnm[j] = jnp.maximum(mnm[j], mn)
            return tuple(mnm)

        def win_body(i, carry):
            war = list(carry[0])
            wix = list(carry[1])
            if win_off is None:
                bi = plsc.load_gather(
                    glist_v, [jnp.full((_LANES,), lev * _NB, jnp.int32) + i])
            else:
                bi = plsc.load_gather(
                    blist_v, [jnp.full((_LANES,), win_off, jnp.int32) + i])
            qv = bi * 4 + jnp.full((_LANES,), bbase, jnp.int32)
            x1 = plsc.load_gather(boxes_v, [qv])
            y1 = plsc.load_gather(boxes_v, [qv + 1])
            x2 = plsc.load_gather(boxes_v, [qv + 2])
            y2 = plsc.load_gather(boxes_v, [qv + 3])
            ar = plsc.load_gather(areas_v, [bi])
            fi = bi.astype(jnp.float32)
            tb_of = mints_of(y1, y2)
            for j in range(nchunks):
                mint, maxt = tb_of(j)
                l = cxs[j] - x1
                rr = x2 - cxs[j]
                mn = jnp.minimum(jnp.minimum(l, rr), mint)
                mx = jnp.maximum(jnp.maximum(l, rr), maxt)
                pred = (mn >= 0.0) & (mx > th0) & (mx <= th1)
                better = pred & (ar <= war[j])
                war[j] = jnp.where(better, ar, war[j])
                wix[j] = jnp.where(better, fi, wix[j])
            return (tuple(war), tuple(wix))

        neg1 = jnp.full((_LANES,), -1.0, jnp.float32)
        big = jnp.full((_LANES,), 3.0e38, jnp.float32)
        fg_n = _NB if fg_src is None else fg_cnt
        mnm = lax.fori_loop(0, fg_n, fg_body,
                            tuple(neg1 for _ in range(nchunks)))
        _, wix = lax.fori_loop(
            0, gcnt, win_body,
            (tuple(big for _ in range(nchunks)),
             tuple(neg1 for _ in range(nchunks))))
        for j in range(nchunks):
            d = pl.ds((woff + j) * _LANES, _LANES)
            win_v[d] = wix[j]
            mn_v[d] = mnm[j]

    geps = 1.0e-5

    def build_band_lists(img, rfirst, lev):
        scale = float(_STRIDES[lev]) / _IMG
        th0, th1 = _THS[lev]
        cyf = (rfirst.astype(jnp.float32) + 0.5) * scale
        cyl = ((rfirst + 7).astype(jnp.float32) + 0.5) * scale
        fcnt = jnp.int32(0)
        wcnt = jnp.int32(0)
        for k in range(_NB // _LANES):
            qb = img * (_NB * 4) + k * _LANES * 4
            qv = iota4 + jnp.full((_LANES,), qb, jnp.int32)
            x1v = plsc.load_gather(boxes_v, [qv])
            y1v = plsc.load_gather(boxes_v, [qv + 1])
            x2v = plsc.load_gather(boxes_v, [qv + 2])
            y2v = plsc.load_gather(boxes_v, [qv + 3])
            yok = ((cyl - y1v) >= 0.0) & ((y2v - cyf) >= 0.0)
            mwh = jnp.maximum(x2v - x1v, y2v - y1v)
            wok = yok & (mwh > th0 - geps) & (mwh <= 2.0 * th1 + geps)
            plsc.store_compressed(blist_v.at[pl.ds(fcnt, _LANES)],
                                  iota + k * _LANES, mask=yok)
            fcnt = fcnt + plsc.all_reduce_population_count(yok)[0]
            plsc.store_compressed(blist_v.at[pl.ds(80 + wcnt, _LANES)],
                                  iota + k * _LANES, mask=wok)
            wcnt = wcnt + plsc.all_reduce_population_count(wok)[0]
        return fcnt, wcnt

    def emit_chunks(img, band, lev, nchunks, store):
        bbase = img * (_NB * 4)

        def ebody(c, carry):
            d = pl.ds(c * _LANES, _LANES)
            widxf = win_v[d]
            mnmax = mn_v[d]
            cx, cy = pix_coords(lev, band + c * _LANES + iota)
            anyfg = mnmax >= 0.0
            haswin = widxf >= 0.0
            wi = jnp.where(haswin, widxf, 0.0).astype(jnp.int32)
            wq = wi * 4 + jnp.full((_LANES,), bbase, jnp.int32)
            x1w = plsc.load_gather(boxes_v, [wq])
            y1w = plsc.load_gather(boxes_v, [wq + 1])
            x2w = plsc.load_gather(boxes_v, [wq + 2])
            y2w = plsc.load_gather(boxes_v, [wq + 3])
            labw = plsc.load_gather(
                labels_v, [wi + jnp.full((_LANES,), img * _NB, jnp.int32)])
            lr = cx - x1w
            tr = cy - y1w
            rr = x2w - cx
            br = y2w - cy
            arg = ((jnp.minimum(lr, rr) / jnp.maximum(lr, rr)) *
                   (jnp.minimum(tr, br) / jnp.maximum(tr, br)))
            pos = haswin & (arg > 0.0)
            safe = jnp.where(pos, arg, 1.0)
            cen = jnp.where(pos, _sqrt_nr(safe), 0.0)
            labm = jnp.where(haswin, labw, 0)

            store(0, c, jnp.where(haswin, lr, 0.0))
            store(1, c, jnp.where(haswin, tr, 0.0))
            store(2, c, jnp.where(haswin, rr, 0.0))
            store(3, c, jnp.where(haswin, br, 0.0))
            store(4, c, cen)
            store(5, c, jnp.where(anyfg, 0.0, 1.0))
            for ch in range(1, _NCLS):
                store(5 + ch, c, jnp.where(labm == ch, 1.0, 0.0))
            return carry

        lax.fori_loop(0, nchunks, ebody, 0)

    def img_body(img, _):
        mwhs = []
        for k in range(_NB // _LANES):
            qb = img * (_NB * 4) + k * _LANES * 4
            qv = iota4 + jnp.full((_LANES,), qb, jnp.int32)
            x1v = plsc.load_gather(boxes_v, [qv])
            y1v = plsc.load_gather(boxes_v, [qv + 1])
            x2v = plsc.load_gather(boxes_v, [qv + 2])
            y2v = plsc.load_gather(boxes_v, [qv + 3])
            areas_v[pl.ds(k * _LANES, _LANES)] = (x2v - x1v) * (y2v - y1v)
            mwhs.append(jnp.maximum(x2v - x1v, y2v - y1v))

        gcnts = {}
        for lev in (3, 4):
            th0, th1 = _THS[lev]
            off = jnp.int32(0)
            for k in range(_NB // _LANES):
                m = (mwhs[k] > th0 - geps) & (mwhs[k] <= 2.0 * th1 + geps)
                plsc.store_compressed(
                    glist_v.at[pl.ds(lev * _NB + off, _LANES)],
                    iota + k * _LANES, mask=m)
                off = off + plsc.all_reduce_population_count(m)[0]
            gcnts[lev] = off

        b0 = jnp.remainder(wid + 24 * img, 32)

        @pl.when(b0 < 8)
        def _():
            band = b0 * 512
            fcnt, wcnt = build_band_lists(img, b0 * 8, 0)

            def blk(kb, carry):
                scan_block(img, band + kb * 128, 0, 8, kb * 8, wcnt,
                           fg_src=0, fg_cnt=fcnt, win_off=80)
                return carry
            lax.fori_loop(0, 4, blk, 0)

            def st0(ch, c, v):
                stage0[ch, lax.shift_right_logical(c, 2),
                       pl.ds(jnp.bitwise_and(c, 3) * _LANES, _LANES)] = v
            emit_chunks(img, band, 0, 32, st0)
            pltpu.sync_copy(stage0, o0.at[img, :, pl.ds(b0 * 8, 8), :])

        b1 = jnp.remainder(wid + 28 * img, 32)

        @pl.when(b1 < 4)
        def _():
            band = b1 * 256
            fcnt, wcnt = build_band_lists(img, b1 * 8, 1)

            def blk(kb, carry):
                scan_block(img, band + kb * 128, 1, 8, kb * 8, wcnt,
                           fg_src=0, fg_cnt=fcnt, win_off=80)
                return carry
            lax.fori_loop(0, 2, blk, 0)

            def st1(ch, c, v):
                stage1[ch, lax.shift_right_logical(c, 1),
                       pl.ds(jnp.bitwise_and(c, 1) * _LANES, _LANES)] = v
            emit_chunks(img, band, 1, 16, st1)
            pltpu.sync_copy(stage1, o1.at[img, :, pl.ds(b1 * 8, 8), :])

        b2 = jnp.remainder(wid + 30 * img, 32)

        @pl.when(b2 < 2)
        def _():
            band = b2 * 128
            fcnt, wcnt = build_band_lists(img, b2 * 8, 2)
            scan_block(img, band, 2, 8, 0, wcnt,
                       fg_src=0, fg_cnt=fcnt, win_off=80)

            def st2(ch, c, v):
                stage2[ch, c, pl.ds(0, _LANES)] = v
            emit_chunks(img, band, 2, 8, st2)
            pltpu.sync_copy(stage2, o2.at[img, :, pl.ds(b2 * 8, 8), :])

        @pl.when(wid == 16 + img)
        def _():
            scan_block(img, 0, 3, 4, 0, gcnts[3])

            def st3(ch, c, v):
                stage3[ch, pl.ds(c * _LANES, _LANES)] = v
            emit_chunks(img, 0, 3, 4, st3)
            pltpu.sync_copy(stage3, o3.at[img])

        @pl.when(wid == 24 + img)
        def _():
            scan_block(img, 0, 4, 1, 0, gcnts[4])

            def st4(ch, c, v):
                stage4[ch, pl.ds(0, _LANES)] = v
            emit_chunks(img, 0, 4, 1, st4)
            pltpu.sync_copy(stage4, o4.at[img])

        return 0

    lax.fori_loop(0, _B, img_body, 0)


def kernel(boxes, labels):
    bflat = boxes.reshape(-1)
    lflat = labels.reshape(-1)

    out_type = (
        jax.ShapeDtypeStruct((_B, _NCH, 64, 64), jnp.float32),
        jax.ShapeDtypeStruct((_B, _NCH, 32, 32), jnp.float32),
        jax.ShapeDtypeStruct((_B, _NCH, 16, 16), jnp.float32),
        jax.ShapeDtypeStruct((_B, _NCH, 64), jnp.float32),
        jax.ShapeDtypeStruct((_B, _NCH, 16), jnp.float32),
    )

    mesh = plsc.VectorSubcoreMesh(core_axis_name="c", subcore_axis_name="s",
                                  num_cores=_NC, num_subcores=_NS)
    run = pl.kernel(
        _sc_mapper,
        out_type=out_type,
        mesh=mesh,
        compiler_params=pltpu.CompilerParams(needs_layout_passes=False),
        scratch_types=[
            pltpu.VMEM((_B * _NB * 4,), jnp.float32),
            pltpu.VMEM((_B * _NB,), jnp.int32),
            pltpu.VMEM((_NB,), jnp.float32),
            pltpu.VMEM((_NCH, 8, 64), jnp.float32),
            pltpu.VMEM((_NCH, 8, 32), jnp.float32),
            pltpu.VMEM((_NCH, 8, 16), jnp.float32),
            pltpu.VMEM((_NCH, 64), jnp.float32),
            pltpu.VMEM((_NCH, 16), jnp.float32),
            pltpu.VMEM((512,), jnp.float32),
            pltpu.VMEM((512,), jnp.float32),
            pltpu.VMEM((5 * _NB + _LANES,), jnp.int32),
            pltpu.VMEM((160,), jnp.int32),
        ],
    )
    o0, o1, o2, o3, o4 = run(bflat, lflat)
    return (o0, o1, o2,
            o3.reshape(_B, _NCH, 8, 8),
            o4.reshape(_B, _NCH, 4, 4))
